# Initial kernel scaffold; baseline (speedup 1.0000x reference)
import jax
import jax.numpy as jnp
from jax.experimental import pallas as pl


def kernel(x, Wg, W1, b1, W2, b2):
    raise NotImplementedError("write your pallas kernel here")


# speedup vs baseline: 1.1812x; 1.1812x over previous
import functools

import jax
import jax.numpy as jnp
from jax import lax
from jax.experimental import pallas as pl
from jax.experimental.pallas import tpu as pltpu
from jax.experimental.pallas import tpu_sc as plsc

S, D, F, E, TOPK = 2048, 1024, 2048, 8, 2
P = S * TOPK
BM = 256
NT = P // BM + E
NPAD = NT * BM

NC, NS = 2, 16
NW = NC * NS


def _router_body(gl_ref, dst_ref, w_ref, te_ref, tv_ref):
    gl = gl_ref[...].T
    eio = lax.broadcasted_iota(jnp.int32, (E, S), 0)
    v1 = jnp.max(gl, axis=0, keepdims=True)
    a1 = jnp.min(jnp.where(gl == v1, eio, E), axis=0, keepdims=True)
    masked = jnp.where(eio == a1, -jnp.inf, gl)
    v2 = jnp.max(masked, axis=0, keepdims=True)
    a2 = jnp.min(jnp.where(masked == v2, eio, E), axis=0, keepdims=True)
    e1 = jnp.exp(v1 - jnp.max(v1))
    e2 = jnp.exp(v2 - jnp.max(v2))
    w_ref[0:1, :] = e1 / jnp.sum(e1)
    w_ref[1:2, :] = e2 / jnp.sum(e2)
    oh1 = (eio == a1).astype(jnp.float32)
    oh2 = (eio == a2).astype(jnp.float32)
    sio_r = lax.broadcasted_iota(jnp.int32, (S, S), 0)
    sio_c = lax.broadcasted_iota(jnp.int32, (S, S), 1)
    incl = (sio_r <= sio_c).astype(jnp.float32)
    cc1 = lax.dot_general(oh1, incl, (((1,), (0,)), ((), ())),
                          precision=lax.Precision.HIGHEST,
                          preferred_element_type=jnp.float32)
    cc2 = lax.dot_general(oh2, incl, (((1,), (0,)), ((), ())),
                          precision=lax.Precision.HIGHEST,
                          preferred_element_type=jnp.float32)
    r1 = jnp.sum(jnp.where(eio == a1, cc1, 0.0), axis=0, keepdims=True) - 1.0
    r2 = jnp.sum(jnp.where(eio == a2, cc2, 0.0), axis=0, keepdims=True) - 1.0
    cnt1 = cc1[:, S - 1:S]
    cnt2 = cc2[:, S - 1:S]
    g = (cnt1 + cnt2).astype(jnp.int32)
    gpad = ((g + (BM - 1)) // BM) * BM
    rio = lax.broadcasted_iota(jnp.int32, (E, E), 0)
    cio = lax.broadcasted_iota(jnp.int32, (E, E), 1)
    tri = (cio < rio).astype(jnp.float32)
    off_f = lax.dot_general(tri, gpad.astype(jnp.float32),
                            (((1,), (0,)), ((), ())),
                            precision=lax.Precision.HIGHEST,
                            preferred_element_type=jnp.float32)
    off = off_f.astype(jnp.int32)
    off1 = jnp.sum(jnp.where(eio == a1, off_f, 0.0), axis=0, keepdims=True)
    off2 = jnp.sum(jnp.where(eio == a2, off_f, 0.0), axis=0, keepdims=True)
    c1a2 = jnp.sum(jnp.where(eio == a2, cnt1, 0.0), axis=0, keepdims=True)
    dst_ref[0:1, :] = (off1 + r1).astype(jnp.int32)
    dst_ref[1:2, :] = (off2 + c1a2 + r2).astype(jnp.int32)
    total = jnp.sum(gpad)
    tio = lax.broadcasted_iota(jnp.int32, (E, NT), 1) * BM
    te_raw = jnp.sum((tio >= off).astype(jnp.int32), axis=0, keepdims=True) - 1
    tvalid = (lax.broadcasted_iota(jnp.int32, (1, NT), 1) * BM) < total
    e8 = lax.broadcasted_iota(jnp.int32, (E, 1), 0)
    emax = jnp.max(jnp.where(gpad > 0, e8, 0))
    te_ref[...] = jnp.where(tvalid, te_raw, emax)
    tv_ref[...] = tvalid.astype(jnp.int32)


def _router(gl):
    return pl.pallas_call(
        _router_body,
        out_shape=[
            jax.ShapeDtypeStruct((TOPK, S), jnp.int32),
            jax.ShapeDtypeStruct((TOPK, S), jnp.float32),
            jax.ShapeDtypeStruct((1, NT), jnp.int32),
            jax.ShapeDtypeStruct((1, NT), jnp.int32),
        ],
    )(gl)


_MESH = plsc.VectorSubcoreMesh(core_axis_name="c", subcore_axis_name="s",
                               num_cores=NC, num_subcores=NS)
_PW = P // NW
_CH = 32


@functools.partial(
    pl.kernel,
    out_type=[
        jax.ShapeDtypeStruct((NPAD, D), jnp.float32),
        jax.ShapeDtypeStruct((NPAD,), jnp.float32),
    ],
    mesh=_MESH,
    scratch_types=[
        pltpu.VMEM((_CH,), jnp.int32),
        pltpu.VMEM((_CH,), jnp.float32),
        pltpu.VMEM((_CH, D), jnp.float32),
        pltpu.SemaphoreType.DMA,
        pltpu.SemaphoreType.DMA,
    ],
)
def _dispatch(x_hbm, dst_hbm, w_hbm, xs_hbm, wd_hbm, idx_v, wv, rows_v,
              sem, semw):
    wid = lax.axis_index("s") * NC + lax.axis_index("c")
    for j in range(_PW // _CH):
        p0 = wid * _PW + j * _CH
        t0 = lax.rem(p0, S)
        pltpu.sync_copy(x_hbm.at[pl.ds(t0, _CH)], rows_v)
        pltpu.sync_copy(dst_hbm.at[pl.ds(p0, _CH)], idx_v)
        pltpu.sync_copy(w_hbm.at[pl.ds(p0, _CH)], wv)
        cpr = pltpu.async_copy(rows_v, xs_hbm.at[idx_v], sem)
        cpw = pltpu.async_copy(wv, wd_hbm.at[idx_v], semw)
        cpr.wait()
        cpw.wait()


def _gmm_body(te_ref, tv_ref, xs_ref, w1_ref, b1_ref, w2_ref, b2_ref, wd_ref,
              ys_ref):
    i = pl.program_id(0)

    @pl.when(tv_ref[i] != 0)
    def _():
        xs = xs_ref[...]
        h = lax.dot_general(xs, w1_ref[0], (((1,), (1,)), ((), ())),
                            preferred_element_type=jnp.float32)
        h = jnp.maximum(h + b1_ref[0], 0.0)
        ys = lax.dot_general(h, w2_ref[0], (((1,), (1,)), ((), ())),
                             preferred_element_type=jnp.float32)
        wd = wd_ref[0]
        ys_ref[...] = (ys + b2_ref[0]) * wd.reshape(BM, 1)


def _gmm(te, tv, xs, W1, b1, W2, b2, wd):
    grid_spec = pltpu.PrefetchScalarGridSpec(
        num_scalar_prefetch=2,
        grid=(NT,),
        in_specs=[
            pl.BlockSpec((BM, D), lambda i, te, tv: (i, 0)),
            pl.BlockSpec((1, F, D), lambda i, te, tv: (te[i], 0, 0)),
            pl.BlockSpec((1, 1, F), lambda i, te, tv: (te[i], 0, 0)),
            pl.BlockSpec((1, D, F), lambda i, te, tv: (te[i], 0, 0)),
            pl.BlockSpec((1, 1, D), lambda i, te, tv: (te[i], 0, 0)),
            pl.BlockSpec((1, 1, BM), lambda i, te, tv: (i, 0, 0)),
        ],
        out_specs=pl.BlockSpec((BM, D), lambda i, te, tv: (i, 0)),
    )
    return pl.pallas_call(
        _gmm_body,
        grid_spec=grid_spec,
        out_shape=jax.ShapeDtypeStruct((NPAD, D), jnp.float32),
    )(te, tv, xs, W1, b1.reshape(E, 1, F), W2, b2.reshape(E, 1, D),
      wd.reshape(NT, 1, BM))


_TK = S // NW
_TC = 16
_NV = D // 16


@functools.partial(
    pl.kernel,
    out_type=jax.ShapeDtypeStruct((S, D), jnp.float32),
    mesh=_MESH,
    scratch_types=[
        pltpu.VMEM((_TC,), jnp.int32),
        pltpu.VMEM((_TC,), jnp.int32),
        pltpu.VMEM((_TC, D), jnp.float32),
        pltpu.VMEM((_TC, D), jnp.float32),
        pltpu.SemaphoreType.DMA,
        pltpu.SemaphoreType.DMA,
    ],
)
def _combine(ys_hbm, d1_hbm, d2_hbm, out_hbm, i1v, i2v, g1, g2, sem1, sem2):
    wid = lax.axis_index("s") * NC + lax.axis_index("c")
    for c in range(_TK // _TC):
        t0 = wid * _TK + c * _TC
        pltpu.sync_copy(d1_hbm.at[pl.ds(t0, _TC)], i1v)
        pltpu.sync_copy(d2_hbm.at[pl.ds(t0, _TC)], i2v)
        cp1 = pltpu.async_copy(ys_hbm.at[i1v], g1, sem1)
        cp2 = pltpu.async_copy(ys_hbm.at[i2v], g2, sem2)
        cp1.wait()
        cp2.wait()

        def vec_body(q, carry):
            j = q // _NV
            sl = pl.ds(lax.rem(q, _NV) * 16, 16)
            g1[j, sl] = g1[j, sl] + g2[j, sl]
            return carry

        lax.fori_loop(0, _TC * _NV, vec_body, 0)
        pltpu.sync_copy(g1, out_hbm.at[pl.ds(t0, _TC)])


def kernel(x, Wg, W1, b1, W2, b2):
    x2 = x.reshape(S, D)
    gl = (x @ Wg.T).reshape(S, E)
    dst, w, te, tv = _router(gl)
    xs, wd = _dispatch(x2, dst.reshape(P), w.reshape(P))
    ys = _gmm(te.reshape(NT), tv.reshape(NT), xs, W1, b1, W2, b2, wd)
    out2 = _combine(ys, dst[0], dst[1])
    return out2.reshape(1, S, D)
